# baseline (speedup 1.0000x reference)
import functools

import jax
import jax.numpy as jnp
from jax import lax
from jax.experimental import pallas as pl
from jax.experimental.pallas import tpu as pltpu
from jax.experimental.pallas import tpu_sc as plsc

_NC = 2
_NS = 16
_NW = _NC * _NS
_L = 16

_BATCH = 16384
_BPW = _BATCH // _NW
_F = 32
_GROUPS = _BPW // _L


def _sc_body(user_hbm, movie_hbm, uf_hbm, mf_hbm, ub_hbm, mb_hbm, gb_hbm,
             out_hbm,
             idx_u, idx_m, urows, mrows, ubv, mbv, gbv, outv, pbuf,
             sem_u, sem_m, sem_ub, sem_mb):
  wid = lax.axis_index("s") * _NC + lax.axis_index("c")
  base = wid * _BPW

  pltpu.sync_copy(user_hbm.at[pl.ds(base, _BPW)], idx_u)
  pltpu.sync_copy(movie_hbm.at[pl.ds(base, _BPW)], idx_m)
  pltpu.sync_copy(gb_hbm, gbv)

  cu = pltpu.async_copy(uf_hbm.at[idx_u], urows, sem_u)
  cm = pltpu.async_copy(mf_hbm.at[idx_m], mrows, sem_m)
  cub = pltpu.async_copy(ub_hbm.at[idx_u], ubv, sem_ub)
  cmb = pltpu.async_copy(mb_hbm.at[idx_m], mbv, sem_mb)
  cu.wait()
  cm.wait()
  cub.wait()
  cmb.wait()

  iota = lax.iota(jnp.int32, _L)
  gb = gbv[...]

  def r_body(i, carry):
    p = (urows[i, pl.ds(0, _L)] * mrows[i, pl.ds(0, _L)] +
         urows[i, pl.ds(_L, _L)] * mrows[i, pl.ds(_L, _L)])
    pbuf[pl.ds(i * _L, _L)] = p
    return carry

  lax.fori_loop(0, _BPW, r_body, 0)

  def g_body(g, carry):
    rows = g * _L + iota
    acc = ubv[pl.ds(g * _L, _L)] + mbv[pl.ds(g * _L, _L)] + gb
    for j in range(_L):
      acc = acc + plsc.load_gather(pbuf, [rows * _L + j])
    outv[pl.ds(g * _L, _L)] = acc
    return carry

  lax.fori_loop(0, _GROUPS, g_body, 0)

  pltpu.sync_copy(outv, out_hbm.at[pl.ds(base, _BPW)])


@jax.jit
def _mf_sc(user, movie, user_factors, movie_factors, ub, mb, gb16):
  fn = pl.kernel(
      _sc_body,
      out_type=jax.ShapeDtypeStruct((_BATCH,), jnp.float32),
      mesh=plsc.VectorSubcoreMesh(core_axis_name="c", subcore_axis_name="s"),
      compiler_params=pltpu.CompilerParams(needs_layout_passes=False,
                                           use_tc_tiling_on_sc=False),
      scratch_types=[
          pltpu.VMEM((_BPW,), jnp.int32),
          pltpu.VMEM((_BPW,), jnp.int32),
          pltpu.VMEM((_BPW, _F), jnp.float32),
          pltpu.VMEM((_BPW, _F), jnp.float32),
          pltpu.VMEM((_BPW,), jnp.float32),
          pltpu.VMEM((_BPW,), jnp.float32),
          pltpu.VMEM((_L,), jnp.float32),
          pltpu.VMEM((_BPW,), jnp.float32),
          pltpu.VMEM((_BPW * _L,), jnp.float32),
          pltpu.SemaphoreType.DMA,
          pltpu.SemaphoreType.DMA,
          pltpu.SemaphoreType.DMA,
          pltpu.SemaphoreType.DMA,
      ],
  )
  return fn(user, movie, user_factors, movie_factors, ub, mb, gb16)


def kernel(user, movie, user_factors, movie_factors, user_bias, movie_bias,
           global_bias):
  user = user.astype(jnp.int32)
  movie = movie.astype(jnp.int32)
  ub = user_bias.reshape(-1)
  mb = movie_bias.reshape(-1)
  gb16 = jnp.broadcast_to(global_bias.astype(jnp.float32), (_L,))
  return _mf_sc(user, movie, user_factors, movie_factors, ub, mb, gb16)


# speedup vs baseline: 1.9855x; 1.9855x over previous
import functools

import jax
import jax.numpy as jnp
from jax import lax
from jax.experimental import pallas as pl
from jax.experimental.pallas import tpu as pltpu
from jax.experimental.pallas import tpu_sc as plsc

_NC = 2
_NS = 16
_NW = _NC * _NS
_L = 16

_BATCH = 16384
_BPW = _BATCH // _NW
_F = 32
_SG = 4
_CHUNK = _L


def _sc_body(user_hbm, movie_hbm, uft_hbm, mft_hbm, ub_hbm, mb_hbm, gb_hbm,
             out_hbm, *refs):
  idx_u, idx_m, gbv, outv = refs[0], refs[1], refs[2], refs[3]
  sets = []
  off = 4
  for s in range(2):
    ublks = refs[off:off + _SG]
    mblks = refs[off + _SG:off + 2 * _SG]
    ubbs = refs[off + 2 * _SG:off + 3 * _SG]
    mbbs = refs[off + 3 * _SG:off + 4 * _SG]
    sem = refs[off + 4 * _SG]
    sets.append((ublks, mblks, ubbs, mbbs, sem))
    off += 4 * _SG + 1

  wid = lax.axis_index("s") * _NC + lax.axis_index("c")
  base = wid * _BPW

  pltpu.sync_copy(user_hbm.at[pl.ds(base, _BPW)], idx_u)
  pltpu.sync_copy(movie_hbm.at[pl.ds(base, _BPW)], idx_m)
  pltpu.sync_copy(gb_hbm, gbv)

  iota = lax.iota(jnp.int32, _L)
  z32 = jnp.zeros((_L,), jnp.int32)
  zf = jnp.zeros((_L,), jnp.float32)
  gb0 = gbv[...][0]

  def issue(vu, vm, sg, st):
    ublks, mblks, ubbs, mbbs, sem = st
    for e in range(_SG):
      cu = vu[sg * _SG + e]
      cm = vm[sg * _SG + e]
      bau = pl.multiple_of((cu >> 7) << 7, 128)
      bam = pl.multiple_of((cm >> 7) << 7, 128)
      pltpu.async_copy(uft_hbm.at[:, pl.ds(bau, 128)], ublks[e], sem)
      pltpu.async_copy(mft_hbm.at[:, pl.ds(bam, 128)], mblks[e], sem)
      pltpu.async_copy(ub_hbm.at[:, pl.ds(bau, 128)], ubbs[e], sem)
      pltpu.async_copy(mb_hbm.at[:, pl.ds(bam, 128)], mbbs[e], sem)

  def compute(c, vu, vm, sg, st):
    ublks, mblks, ubbs, mbbs, sem = st
    for e in range(_SG):
      pltpu.make_async_copy(uft_hbm.at[:, pl.ds(0, 128)], ublks[e], sem).wait()
      pltpu.make_async_copy(mft_hbm.at[:, pl.ds(0, 128)], mblks[e], sem).wait()
      pltpu.make_async_copy(ub_hbm.at[:, pl.ds(0, 128)], ubbs[e], sem).wait()
      pltpu.make_async_copy(mb_hbm.at[:, pl.ds(0, 128)], mbbs[e], sem).wait()
    for e in range(_SG):
      cu = vu[sg * _SG + e]
      cm = vm[sg * _SG + e]
      lu = z32 + (cu & 127)
      lm = z32 + (cm & 127)
      u0 = plsc.load_gather(ublks[e], [iota, lu])
      u1 = plsc.load_gather(ublks[e], [iota + _L, lu])
      m0 = plsc.load_gather(mblks[e], [iota, lm])
      m1 = plsc.load_gather(mblks[e], [iota + _L, lm])
      ubx = plsc.load_gather(ubbs[e], [z32, lu])
      mbx = plsc.load_gather(mbbs[e], [z32, lm])
      total = jnp.sum(u0 * m0 + u1 * m1) + ubx[0] + mbx[0] + gb0
      i = c * _CHUNK + sg * _SG + e
      plsc.store_scatter(outv, [z32 + i], zf + total, mask=iota == 0)

  def c_body(c, carry):
    vu = idx_u[pl.ds(c * _CHUNK, _CHUNK)]
    vm = idx_m[pl.ds(c * _CHUNK, _CHUNK)]
    issue(vu, vm, 0, sets[0])
    issue(vu, vm, 1, sets[1])
    compute(c, vu, vm, 0, sets[0])
    issue(vu, vm, 2, sets[0])
    compute(c, vu, vm, 1, sets[1])
    issue(vu, vm, 3, sets[1])
    compute(c, vu, vm, 2, sets[0])
    compute(c, vu, vm, 3, sets[1])
    return carry

  lax.fori_loop(0, _BPW // _CHUNK, c_body, 0)

  pltpu.sync_copy(outv, out_hbm.at[pl.ds(base, _BPW)])


@jax.jit
def _mf_sc(user, movie, uft, mft, ubt, mbt, gb16):
  scratch = [
      pltpu.VMEM((_BPW,), jnp.int32),
      pltpu.VMEM((_BPW,), jnp.int32),
      pltpu.VMEM((_L,), jnp.float32),
      pltpu.VMEM((_BPW,), jnp.float32),
  ]
  for s in range(2):
    scratch += [pltpu.VMEM((_F, 128), jnp.float32)] * _SG
    scratch += [pltpu.VMEM((_F, 128), jnp.float32)] * _SG
    scratch += [pltpu.VMEM((1, 128), jnp.float32)] * _SG
    scratch += [pltpu.VMEM((1, 128), jnp.float32)] * _SG
    scratch += [pltpu.SemaphoreType.DMA]
  fn = pl.kernel(
      _sc_body,
      out_type=jax.ShapeDtypeStruct((_BATCH,), jnp.float32),
      mesh=plsc.VectorSubcoreMesh(core_axis_name="c", subcore_axis_name="s"),
      compiler_params=pltpu.CompilerParams(needs_layout_passes=False,
                                           use_tc_tiling_on_sc=True),
      scratch_types=scratch,
  )
  return fn(user, movie, uft, mft, ubt, mbt, gb16)


def kernel(user, movie, user_factors, movie_factors, user_bias, movie_bias,
           global_bias):
  user = user.astype(jnp.int32)
  movie = movie.astype(jnp.int32)
  uft = user_factors.T
  mft = movie_factors.T
  ubt = user_bias.T
  mbt = movie_bias.T
  gb16 = jnp.broadcast_to(global_bias.astype(jnp.float32), (_L,))
  return _mf_sc(user, movie, uft, mft, ubt, mbt, gb16)


# speedup vs baseline: 2.6688x; 1.3442x over previous
import functools

import jax
import jax.numpy as jnp
from jax import lax
from jax.experimental import pallas as pl
from jax.experimental.pallas import tpu as pltpu
from jax.experimental.pallas import tpu_sc as plsc

_NC = 2
_NS = 16
_NW = _NC * _NS
_L = 16

_BATCH = 16384
_BPW = _BATCH // _NW
_F = 32
_SG = 4
_NSETS = 3
_CHUNK = _L
_GROUPS = _BPW // _L


def _gather_u_body(user_hbm, uft_hbm, uout_hbm, *refs):
  idx_u, urows = refs[0], refs[1]
  sets = []
  off = 2
  for s in range(_NSETS):
    sets.append((refs[off:off + _SG], refs[off + _SG]))
    off += _SG + 1

  wid = lax.axis_index("s") * _NC + lax.axis_index("c")
  base = wid * _BPW

  pltpu.sync_copy(user_hbm.at[pl.ds(base, _BPW)], idx_u)

  iota = lax.iota(jnp.int32, _L)
  z32 = jnp.zeros((_L,), jnp.int32)

  def issue(vu, sg, st):
    ublks, sem = st
    for e in range(_SG):
      cu = vu[(sg % _SG) * _SG + e]
      ba = pl.multiple_of((cu >> 7) << 7, 128)
      pltpu.async_copy(uft_hbm.at[:, pl.ds(ba, 128)], ublks[e], sem)

  def compute(c, vu, sg, st):
    ublks, sem = st
    for e in range(_SG):
      pltpu.make_async_copy(uft_hbm.at[:, pl.ds(0, 128)], ublks[e], sem).wait()
    for e in range(_SG):
      cu = vu[(sg % _SG) * _SG + e]
      lu = z32 + (cu & 127)
      u0 = plsc.load_gather(ublks[e], [iota, lu])
      u1 = plsc.load_gather(ublks[e], [iota + _L, lu])
      i = (c * _CHUNK + (sg % _SG) * _SG + e) * _F
      plsc.store_scatter(urows, [i + iota], u0)
      plsc.store_scatter(urows, [i + _L + iota], u1)

  def c_body(c, carry):
    vu = idx_u[pl.ds(c * _CHUNK, _CHUNK)]
    issue(vu, 0, sets[0])
    issue(vu, 1, sets[1])
    issue(vu, 2, sets[2])
    compute(c, vu, 0, sets[0])
    issue(vu, 3, sets[0])
    compute(c, vu, 1, sets[1])
    compute(c, vu, 2, sets[2])
    compute(c, vu, 3, sets[0])
    return carry

  lax.fori_loop(0, _BPW // _CHUNK, c_body, 0)

  pltpu.sync_copy(urows, uout_hbm.at[pl.ds(base * _F, _BPW * _F)])


def _dot_body(movie_hbm, user_hbm, urows_hbm, mf_hbm, ub_hbm, mb_hbm, gb_hbm,
              out_hbm,
              idx_u, idx_m, urows, mrows, ubv, mbv, gbv, outv, pbuf,
              sem_u, sem_m, sem_ub, sem_mb):
  wid = lax.axis_index("s") * _NC + lax.axis_index("c")
  base = wid * _BPW

  pltpu.sync_copy(user_hbm.at[pl.ds(base, _BPW)], idx_u)
  pltpu.sync_copy(movie_hbm.at[pl.ds(base, _BPW)], idx_m)
  pltpu.sync_copy(gb_hbm, gbv)

  cu = pltpu.async_copy(urows_hbm.at[pl.ds(base * _F, _BPW * _F)], urows,
                        sem_u)
  cm = pltpu.async_copy(mf_hbm.at[idx_m], mrows, sem_m)
  cub = pltpu.async_copy(ub_hbm.at[idx_u], ubv, sem_ub)
  cmb = pltpu.async_copy(mb_hbm.at[idx_m], mbv, sem_mb)
  cu.wait()
  cm.wait()
  cub.wait()
  cmb.wait()

  iota = lax.iota(jnp.int32, _L)
  gb = gbv[...]

  def r_body(i, carry):
    p = (urows[pl.ds(i * _F, _L)] * mrows[i, pl.ds(0, _L)] +
         urows[pl.ds(i * _F + _L, _L)] * mrows[i, pl.ds(_L, _L)])
    pbuf[pl.ds(i * _L, _L)] = p
    return carry

  lax.fori_loop(0, _BPW, r_body, 0)

  def g_body(g, carry):
    rows = g * _L + iota
    acc = ubv[pl.ds(g * _L, _L)] + mbv[pl.ds(g * _L, _L)] + gb
    for j in range(_L):
      acc = acc + plsc.load_gather(pbuf, [rows * _L + j])
    outv[pl.ds(g * _L, _L)] = acc
    return carry

  lax.fori_loop(0, _GROUPS, g_body, 0)

  pltpu.sync_copy(outv, out_hbm.at[pl.ds(base, _BPW)])


@jax.jit
def _mf_sc(user, movie, uft, mf, ub, mb, gb16):
  a_scratch = [
      pltpu.VMEM((_BPW,), jnp.int32),
      pltpu.VMEM((_BPW * _F,), jnp.float32),
  ]
  for s in range(_NSETS):
    a_scratch += [pltpu.VMEM((_F, 128), jnp.float32)] * _SG
    a_scratch += [pltpu.SemaphoreType.DMA]
  gather_u = pl.kernel(
      _gather_u_body,
      out_type=jax.ShapeDtypeStruct((_BATCH * _F,), jnp.float32),
      mesh=plsc.VectorSubcoreMesh(core_axis_name="c", subcore_axis_name="s"),
      compiler_params=pltpu.CompilerParams(needs_layout_passes=False,
                                           use_tc_tiling_on_sc=True),
      scratch_types=a_scratch,
  )
  urows = gather_u(user, uft)

  dot = pl.kernel(
      _dot_body,
      out_type=jax.ShapeDtypeStruct((_BATCH,), jnp.float32),
      mesh=plsc.VectorSubcoreMesh(core_axis_name="c", subcore_axis_name="s"),
      compiler_params=pltpu.CompilerParams(needs_layout_passes=False,
                                           use_tc_tiling_on_sc=False),
      scratch_types=[
          pltpu.VMEM((_BPW,), jnp.int32),
          pltpu.VMEM((_BPW,), jnp.int32),
          pltpu.VMEM((_BPW * _F,), jnp.float32),
          pltpu.VMEM((_BPW, _F), jnp.float32),
          pltpu.VMEM((_BPW,), jnp.float32),
          pltpu.VMEM((_BPW,), jnp.float32),
          pltpu.VMEM((_L,), jnp.float32),
          pltpu.VMEM((_BPW,), jnp.float32),
          pltpu.VMEM((_BPW * _L,), jnp.float32),
          pltpu.SemaphoreType.DMA,
          pltpu.SemaphoreType.DMA,
          pltpu.SemaphoreType.DMA,
          pltpu.SemaphoreType.DMA,
      ],
  )
  return dot(movie, user, urows, mf, ub, mb, gb16)


def kernel(user, movie, user_factors, movie_factors, user_bias, movie_bias,
           global_bias):
  user = user.astype(jnp.int32)
  movie = movie.astype(jnp.int32)
  uft = user_factors.T
  ub = user_bias.reshape(-1)
  mb = movie_bias.reshape(-1)
  gb16 = jnp.broadcast_to(global_bias.astype(jnp.float32), (_L,))
  return _mf_sc(user, movie, uft, movie_factors, ub, mb, gb16)


# speedup vs baseline: 3.1841x; 1.1931x over previous
import functools

import jax
import jax.numpy as jnp
from jax import lax
from jax.experimental import pallas as pl
from jax.experimental.pallas import tpu as pltpu
from jax.experimental.pallas import tpu_sc as plsc

_NC = 2
_NS = 16
_NW = _NC * _NS
_L = 16

_BATCH = 16384
_BPW = _BATCH // _NW
_F = 32
_SG = 8
_NSETS = 3
_CHUNK = _L
_GROUPS = _BPW // _L


def _gather_u_body(user_hbm, uft_hbm, uout_hbm, *refs):
  idx_u, urows = refs[0], refs[1]
  sets = []
  off = 2
  for s in range(_NSETS):
    sets.append((refs[off:off + _SG], refs[off + _SG]))
    off += _SG + 1

  wid = lax.axis_index("s") * _NC + lax.axis_index("c")
  base = wid * _BPW

  pltpu.sync_copy(user_hbm.at[pl.ds(base, _BPW)], idx_u.at[pl.ds(0, _BPW)])

  iota = lax.iota(jnp.int32, _L)
  z32 = jnp.zeros((_L,), jnp.int32)

  def load_half(chunk, parity):
    v = idx_u[pl.ds(chunk * _L, _L)]
    return jnp.minimum(jnp.maximum(v, 0), 999999), parity * _SG

  def issue(chunk, parity, st):
    ublks, sem = st
    vu, lane0 = load_half(chunk, parity)
    for e in range(_SG):
      cu = vu[lane0 + e]
      ba = pl.multiple_of((cu >> 7) << 7, 128)
      pltpu.async_copy(uft_hbm.at[:, pl.ds(ba, 128)], ublks[e], sem)

  def compute(sg, chunk, parity, st):
    ublks, sem = st
    vu, lane0 = load_half(chunk, parity)
    for e in range(_SG):
      pltpu.make_async_copy(uft_hbm.at[:, pl.ds(0, 128)], ublks[e], sem).wait()
    for e in range(_SG):
      cu = vu[lane0 + e]
      lu = z32 + (cu & 127)
      u0 = plsc.load_gather(ublks[e], [iota, lu])
      u1 = plsc.load_gather(ublks[e], [iota + _L, lu])
      i = (sg * _SG + e) * _F
      plsc.store_scatter(urows, [i + iota], u0)
      plsc.store_scatter(urows, [i + _L + iota], u1)

  nsg = _BPW // _SG
  issue(0, 0, sets[0])
  issue(0, 1, sets[1])
  issue(1, 0, sets[2])

  def u_body(s, carry):
    for k in range(6):
      sg = 6 * s + k
      compute(sg, 3 * s + k // 2, k % 2, sets[k % _NSETS])
      issue(3 * s + (k + _NSETS) // 2, (k + _NSETS) % 2, sets[k % _NSETS])
    return carry

  lax.fori_loop(0, 10, u_body, 0)
  for k in range(60, nsg):
    compute(k, k // 2, k % 2, sets[k % _NSETS])
    if k + _NSETS < nsg:
      issue((k + _NSETS) // 2, (k + _NSETS) % 2, sets[k % _NSETS])

  pltpu.sync_copy(urows, uout_hbm.at[pl.ds(base * _F, _BPW * _F)])


def _dot_body(movie_hbm, user_hbm, urows_hbm, mf_hbm, ub_hbm, mb_hbm, gb_hbm,
              out_hbm,
              idx_u, idx_m, urows, mrows, ubv, mbv, gbv, outv, pbuf,
              sem_u, sem_m, sem_ub, sem_mb):
  wid = lax.axis_index("s") * _NC + lax.axis_index("c")
  base = wid * _BPW

  pltpu.sync_copy(user_hbm.at[pl.ds(base, _BPW)], idx_u)
  pltpu.sync_copy(movie_hbm.at[pl.ds(base, _BPW)], idx_m)
  pltpu.sync_copy(gb_hbm, gbv)

  cu = pltpu.async_copy(urows_hbm.at[pl.ds(base * _F, _BPW * _F)], urows,
                        sem_u)
  cm = pltpu.async_copy(mf_hbm.at[idx_m], mrows, sem_m)
  cub = pltpu.async_copy(ub_hbm.at[idx_u], ubv, sem_ub)
  cmb = pltpu.async_copy(mb_hbm.at[idx_m], mbv, sem_mb)
  cu.wait()
  cm.wait()
  cub.wait()
  cmb.wait()

  iota = lax.iota(jnp.int32, _L)
  gb = gbv[...]

  def r_body(i, carry):
    p = (urows[pl.ds(i * _F, _L)] * mrows[i, pl.ds(0, _L)] +
         urows[pl.ds(i * _F + _L, _L)] * mrows[i, pl.ds(_L, _L)])
    pbuf[pl.ds(i * _L, _L)] = p
    return carry

  lax.fori_loop(0, _BPW, r_body, 0)

  def g_body(g, carry):
    rows = g * _L + iota
    acc = ubv[pl.ds(g * _L, _L)] + mbv[pl.ds(g * _L, _L)] + gb
    for j in range(_L):
      acc = acc + plsc.load_gather(pbuf, [rows * _L + j])
    outv[pl.ds(g * _L, _L)] = acc
    return carry

  lax.fori_loop(0, _GROUPS, g_body, 0)

  pltpu.sync_copy(outv, out_hbm.at[pl.ds(base, _BPW)])


@jax.jit
def _mf_sc(user, movie, uft, mf, ub, mb, gb16):
  a_scratch = [
      pltpu.VMEM((_BPW,), jnp.int32),
      pltpu.VMEM((_BPW * _F,), jnp.float32),
  ]
  for s in range(_NSETS):
    a_scratch += [pltpu.VMEM((_F, 128), jnp.float32)] * _SG
    a_scratch += [pltpu.SemaphoreType.DMA]
  gather_u = pl.kernel(
      _gather_u_body,
      out_type=jax.ShapeDtypeStruct((_BATCH * _F,), jnp.float32),
      mesh=plsc.VectorSubcoreMesh(core_axis_name="c", subcore_axis_name="s"),
      compiler_params=pltpu.CompilerParams(needs_layout_passes=False,
                                           use_tc_tiling_on_sc=True),
      scratch_types=a_scratch,
  )
  urows = gather_u(user, uft)

  dot = pl.kernel(
      _dot_body,
      out_type=jax.ShapeDtypeStruct((_BATCH,), jnp.float32),
      mesh=plsc.VectorSubcoreMesh(core_axis_name="c", subcore_axis_name="s"),
      compiler_params=pltpu.CompilerParams(needs_layout_passes=False,
                                           use_tc_tiling_on_sc=False),
      scratch_types=[
          pltpu.VMEM((_BPW,), jnp.int32),
          pltpu.VMEM((_BPW,), jnp.int32),
          pltpu.VMEM((_BPW * _F,), jnp.float32),
          pltpu.VMEM((_BPW, _F), jnp.float32),
          pltpu.VMEM((_BPW,), jnp.float32),
          pltpu.VMEM((_BPW,), jnp.float32),
          pltpu.VMEM((_L,), jnp.float32),
          pltpu.VMEM((_BPW,), jnp.float32),
          pltpu.VMEM((_BPW * _L,), jnp.float32),
          pltpu.SemaphoreType.DMA,
          pltpu.SemaphoreType.DMA,
          pltpu.SemaphoreType.DMA,
          pltpu.SemaphoreType.DMA,
      ],
  )
  return dot(movie, user, urows, mf, ub, mb, gb16)


def kernel(user, movie, user_factors, movie_factors, user_bias, movie_bias,
           global_bias):
  user = user.astype(jnp.int32)
  movie = movie.astype(jnp.int32)
  uft = user_factors.T
  ub = user_bias.reshape(-1)
  mb = movie_bias.reshape(-1)
  gb16 = jnp.broadcast_to(global_bias.astype(jnp.float32), (_L,))
  return _mf_sc(user, movie, uft, movie_factors, ub, mb, gb16)


# speedup vs baseline: 3.1932x; 1.0029x over previous
import functools

import jax
import jax.numpy as jnp
from jax import lax
from jax.experimental import pallas as pl
from jax.experimental.pallas import tpu as pltpu
from jax.experimental.pallas import tpu_sc as plsc

_NC = 2
_NS = 16
_NW = _NC * _NS
_L = 16

_BATCH = 16384
_BPW = _BATCH // _NW
_F = 32
_SG = 8
_NSETS = 3
_CHUNK = _L
_GROUPS = _BPW // _L


def _gather_u_body(user_hbm, uft_hbm, uout_hbm, *refs):
  idx_u, urows = refs[0], refs[1]
  sets = []
  off = 2
  for s in range(_NSETS):
    sets.append((refs[off:off + _SG], refs[off + _SG]))
    off += _SG + 1

  wid = lax.axis_index("s") * _NC + lax.axis_index("c")
  base = wid * _BPW

  pltpu.sync_copy(user_hbm.at[pl.ds(base, _BPW)], idx_u.at[pl.ds(0, _BPW)])

  iota = lax.iota(jnp.int32, _L)
  z32 = jnp.zeros((_L,), jnp.int32)

  def load_half(chunk, parity):
    v = idx_u[pl.ds(chunk * _L, _L)]
    return jnp.minimum(jnp.maximum(v, 0), 999999), parity * _SG

  def issue(chunk, parity, st):
    ublks, sem = st
    vu, lane0 = load_half(chunk, parity)
    for e in range(_SG):
      cu = vu[lane0 + e]
      ba = pl.multiple_of((cu >> 7) << 7, 128)
      for a in range(_F // 8):
        pltpu.async_copy(uft_hbm.at[pl.ds(8 * a, 8), pl.ds(ba, 128)],
                         ublks[e].at[pl.ds(8 * a, 8), :], sem)

  def compute(sg, chunk, parity, st):
    ublks, sem = st
    vu, lane0 = load_half(chunk, parity)
    for e in range(_SG):
      pltpu.make_async_copy(uft_hbm.at[:, pl.ds(0, 128)], ublks[e], sem).wait()
    for e in range(_SG):
      cu = vu[lane0 + e]
      lu = z32 + (cu & 127)
      u0 = plsc.load_gather(ublks[e], [iota, lu])
      u1 = plsc.load_gather(ublks[e], [iota + _L, lu])
      i = (sg * _SG + e) * _F
      plsc.store_scatter(urows, [i + iota], u0)
      plsc.store_scatter(urows, [i + _L + iota], u1)

  nsg = _BPW // _SG
  issue(0, 0, sets[0])
  issue(0, 1, sets[1])
  issue(1, 0, sets[2])

  def u_body(s, carry):
    for k in range(6):
      sg = 6 * s + k
      compute(sg, 3 * s + k // 2, k % 2, sets[k % _NSETS])
      issue(3 * s + (k + _NSETS) // 2, (k + _NSETS) % 2, sets[k % _NSETS])
    return carry

  lax.fori_loop(0, 10, u_body, 0)
  for k in range(60, nsg):
    compute(k, k // 2, k % 2, sets[k % _NSETS])
    if k + _NSETS < nsg:
      issue((k + _NSETS) // 2, (k + _NSETS) % 2, sets[k % _NSETS])

  pltpu.sync_copy(urows, uout_hbm.at[pl.ds(base * _F, _BPW * _F)])


def _dot_body(movie_hbm, user_hbm, urows_hbm, mf_hbm, ub_hbm, mb_hbm, gb_hbm,
              out_hbm,
              idx_u, idx_m, urows, mrows, ubv, mbv, gbv, outv, pbuf,
              sem_u, sem_m, sem_ub, sem_mb):
  wid = lax.axis_index("s") * _NC + lax.axis_index("c")
  base = wid * _BPW

  pltpu.sync_copy(user_hbm.at[pl.ds(base, _BPW)], idx_u)
  pltpu.sync_copy(movie_hbm.at[pl.ds(base, _BPW)], idx_m)
  pltpu.sync_copy(gb_hbm, gbv)

  cu = pltpu.async_copy(urows_hbm.at[pl.ds(base * _F, _BPW * _F)], urows,
                        sem_u)
  cm = pltpu.async_copy(mf_hbm.at[idx_m], mrows, sem_m)
  cub = pltpu.async_copy(ub_hbm.at[idx_u], ubv, sem_ub)
  cmb = pltpu.async_copy(mb_hbm.at[idx_m], mbv, sem_mb)
  cu.wait()
  cm.wait()
  cub.wait()
  cmb.wait()

  iota = lax.iota(jnp.int32, _L)
  gb = gbv[...]

  def r_body(i, carry):
    p = (urows[pl.ds(i * _F, _L)] * mrows[i, pl.ds(0, _L)] +
         urows[pl.ds(i * _F + _L, _L)] * mrows[i, pl.ds(_L, _L)])
    pbuf[pl.ds(i * _L, _L)] = p
    return carry

  lax.fori_loop(0, _BPW, r_body, 0)

  def g_body(g, carry):
    rows = g * _L + iota
    acc = ubv[pl.ds(g * _L, _L)] + mbv[pl.ds(g * _L, _L)] + gb
    for j in range(_L):
      acc = acc + plsc.load_gather(pbuf, [rows * _L + j])
    outv[pl.ds(g * _L, _L)] = acc
    return carry

  lax.fori_loop(0, _GROUPS, g_body, 0)

  pltpu.sync_copy(outv, out_hbm.at[pl.ds(base, _BPW)])


@jax.jit
def _mf_sc(user, movie, uft, mf, ub, mb, gb16):
  a_scratch = [
      pltpu.VMEM((_BPW,), jnp.int32),
      pltpu.VMEM((_BPW * _F,), jnp.float32),
  ]
  for s in range(_NSETS):
    a_scratch += [pltpu.VMEM((_F, 128), jnp.float32)] * _SG
    a_scratch += [pltpu.SemaphoreType.DMA]
  gather_u = pl.kernel(
      _gather_u_body,
      out_type=jax.ShapeDtypeStruct((_BATCH * _F,), jnp.float32),
      mesh=plsc.VectorSubcoreMesh(core_axis_name="c", subcore_axis_name="s"),
      compiler_params=pltpu.CompilerParams(needs_layout_passes=False,
                                           use_tc_tiling_on_sc=True),
      scratch_types=a_scratch,
  )
  urows = gather_u(user, uft)

  dot = pl.kernel(
      _dot_body,
      out_type=jax.ShapeDtypeStruct((_BATCH,), jnp.float32),
      mesh=plsc.VectorSubcoreMesh(core_axis_name="c", subcore_axis_name="s"),
      compiler_params=pltpu.CompilerParams(needs_layout_passes=False,
                                           use_tc_tiling_on_sc=False),
      scratch_types=[
          pltpu.VMEM((_BPW,), jnp.int32),
          pltpu.VMEM((_BPW,), jnp.int32),
          pltpu.VMEM((_BPW * _F,), jnp.float32),
          pltpu.VMEM((_BPW, _F), jnp.float32),
          pltpu.VMEM((_BPW,), jnp.float32),
          pltpu.VMEM((_BPW,), jnp.float32),
          pltpu.VMEM((_L,), jnp.float32),
          pltpu.VMEM((_BPW,), jnp.float32),
          pltpu.VMEM((_BPW * _L,), jnp.float32),
          pltpu.SemaphoreType.DMA,
          pltpu.SemaphoreType.DMA,
          pltpu.SemaphoreType.DMA,
          pltpu.SemaphoreType.DMA,
      ],
  )
  return dot(movie, user, urows, mf, ub, mb, gb16)


def kernel(user, movie, user_factors, movie_factors, user_bias, movie_bias,
           global_bias):
  user = user.astype(jnp.int32)
  movie = movie.astype(jnp.int32)
  uft = user_factors.T
  ub = user_bias.reshape(-1)
  mb = movie_bias.reshape(-1)
  gb16 = jnp.broadcast_to(global_bias.astype(jnp.float32), (_L,))
  return _mf_sc(user, movie, uft, movie_factors, ub, mb, gb16)


# speedup vs baseline: 3.1999x; 1.0021x over previous
import functools

import jax
import jax.numpy as jnp
from jax import lax
from jax.experimental import pallas as pl
from jax.experimental.pallas import tpu as pltpu
from jax.experimental.pallas import tpu_sc as plsc

_NC = 2
_NS = 16
_NW = _NC * _NS
_L = 16

_BATCH = 16384
_BPW = _BATCH // _NW
_F = 32
_SG = 8
_NSETS = 3
_GROUPS = _BPW // _L


def _gather_u_body(user_hbm, uft_hbm, uout_hbm, *refs):
  idx_u, urows = refs[0], refs[1]
  sets = []
  off = 2
  for s in range(_NSETS):
    sets.append((refs[off:off + _SG], refs[off + _SG]))
    off += _SG + 1

  wid = lax.axis_index("s") * _NC + lax.axis_index("c")
  base = wid * _BPW

  pltpu.sync_copy(user_hbm.at[pl.ds(base, _BPW)], idx_u.at[pl.ds(0, _BPW)])

  iota = lax.iota(jnp.int32, _L)
  z32 = jnp.zeros((_L,), jnp.int32)

  def load_half(chunk, parity):
    v = idx_u[pl.ds(chunk * _L, _L)]
    return jnp.minimum(jnp.maximum(v, 0), 999999), parity * _SG

  def issue(chunk, parity, st):
    ublks, sem = st
    vu, lane0 = load_half(chunk, parity)
    for e in range(_SG):
      cu = vu[lane0 + e]
      ba = pl.multiple_of((cu >> 7) << 7, 128)
      for a in range(_F // 8):
        pltpu.async_copy(uft_hbm.at[pl.ds(8 * a, 8), pl.ds(ba, 128)],
                         ublks[e].at[pl.ds(8 * a, 8), :], sem)

  def compute(sg, chunk, parity, st):
    ublks, sem = st
    vu, lane0 = load_half(chunk, parity)
    for e in range(_SG):
      pltpu.make_async_copy(uft_hbm.at[:, pl.ds(0, 128)], ublks[e], sem).wait()
    for e in range(_SG):
      cu = vu[lane0 + e]
      lu = z32 + (cu & 127)
      u0 = plsc.load_gather(ublks[e], [iota, lu])
      u1 = plsc.load_gather(ublks[e], [iota + _L, lu])
      i = (sg * _SG + e) * _F
      plsc.store_scatter(urows, [i + iota], u0)
      plsc.store_scatter(urows, [i + _L + iota], u1)

  nsg = _BPW // _SG
  issue(0, 0, sets[0])
  issue(0, 1, sets[1])
  issue(1, 0, sets[2])

  def u_body(s, carry):
    for k in range(6):
      sg = 6 * s + k
      compute(sg, 3 * s + k // 2, k % 2, sets[k % _NSETS])
      issue(3 * s + (k + _NSETS) // 2, (k + _NSETS) % 2, sets[k % _NSETS])
    return carry

  lax.fori_loop(0, 10, u_body, 0)
  for k in range(60, nsg):
    compute(k, k // 2, k % 2, sets[k % _NSETS])
    if k + _NSETS < nsg:
      issue((k + _NSETS) // 2, (k + _NSETS) % 2, sets[k % _NSETS])

  pltpu.sync_copy(urows, uout_hbm.at[pl.ds(base * _F, _BPW * _F)])


def _dot_body(movie_hbm, user_hbm, urows_hbm, mf_hbm, ub_hbm, mb_hbm, gb_hbm,
              out_hbm,
              idx_u, idx_m, urows, mrows, ubv, mbv, gbv, outv, pbuf,
              sem_u, sem_m, sem_ub, sem_mb):
  wid = lax.axis_index("s") * _NC + lax.axis_index("c")
  base = wid * _BPW

  pltpu.sync_copy(user_hbm.at[pl.ds(base, _BPW)], idx_u)
  pltpu.sync_copy(movie_hbm.at[pl.ds(base, _BPW)], idx_m)
  pltpu.sync_copy(gb_hbm, gbv)

  cu = pltpu.async_copy(urows_hbm.at[pl.ds(base * _F, _BPW * _F)], urows,
                        sem_u)
  cm = pltpu.async_copy(mf_hbm.at[idx_m], mrows, sem_m)
  cub = pltpu.async_copy(ub_hbm.at[idx_u], ubv, sem_ub)
  cmb = pltpu.async_copy(mb_hbm.at[idx_m], mbv, sem_mb)
  cu.wait()
  cm.wait()
  cub.wait()
  cmb.wait()

  iota = lax.iota(jnp.int32, _L)
  gb = gbv[...]

  def r_body(i, carry):
    p = (urows[pl.ds(i * _F, _L)] * mrows[i, pl.ds(0, _L)] +
         urows[pl.ds(i * _F + _L, _L)] * mrows[i, pl.ds(_L, _L)])
    pbuf[pl.ds(i * _L, _L)] = p
    return carry

  lax.fori_loop(0, _BPW, r_body, 0)

  def g_body(g, carry):
    rows = g * _L + iota
    acc = ubv[pl.ds(g * _L, _L)] + mbv[pl.ds(g * _L, _L)] + gb
    for j in range(_L):
      acc = acc + plsc.load_gather(pbuf, [rows * _L + j])
    outv[pl.ds(g * _L, _L)] = acc
    return carry

  lax.fori_loop(0, _GROUPS, g_body, 0)

  pltpu.sync_copy(outv, out_hbm.at[pl.ds(base, _BPW)])


@jax.jit
def _mf_sc(user, movie, uft, mf, ub, mb, gb16):
  a_scratch = [
      pltpu.VMEM((_BPW,), jnp.int32),
      pltpu.VMEM((_BPW * _F,), jnp.float32),
  ]
  for s in range(_NSETS):
    a_scratch += [pltpu.VMEM((_F, 128), jnp.float32)] * _SG
    a_scratch += [pltpu.SemaphoreType.DMA]
  gather_u = pl.kernel(
      _gather_u_body,
      out_type=jax.ShapeDtypeStruct((_BATCH * _F,), jnp.float32),
      mesh=plsc.VectorSubcoreMesh(core_axis_name="c", subcore_axis_name="s"),
      compiler_params=pltpu.CompilerParams(needs_layout_passes=False,
                                           use_tc_tiling_on_sc=True),
      scratch_types=a_scratch,
  )
  urows = gather_u(user, uft)

  dot = pl.kernel(
      _dot_body,
      out_type=jax.ShapeDtypeStruct((_BATCH,), jnp.float32),
      mesh=plsc.VectorSubcoreMesh(core_axis_name="c", subcore_axis_name="s"),
      compiler_params=pltpu.CompilerParams(needs_layout_passes=False,
                                           use_tc_tiling_on_sc=False),
      scratch_types=[
          pltpu.VMEM((_BPW,), jnp.int32),
          pltpu.VMEM((_BPW,), jnp.int32),
          pltpu.VMEM((_BPW * _F,), jnp.float32),
          pltpu.VMEM((_BPW, _F), jnp.float32),
          pltpu.VMEM((_BPW,), jnp.float32),
          pltpu.VMEM((_BPW,), jnp.float32),
          pltpu.VMEM((_L,), jnp.float32),
          pltpu.VMEM((_BPW,), jnp.float32),
          pltpu.VMEM((_BPW * _L,), jnp.float32),
          pltpu.SemaphoreType.DMA,
          pltpu.SemaphoreType.DMA,
          pltpu.SemaphoreType.DMA,
          pltpu.SemaphoreType.DMA,
      ],
  )
  return dot(movie, user, urows, mf, ub, mb, gb16)


def kernel(user, movie, user_factors, movie_factors, user_bias, movie_bias,
           global_bias):
  user = user.astype(jnp.int32)
  movie = movie.astype(jnp.int32)
  uft = user_factors.T
  ub = user_bias.reshape(-1)
  mb = movie_bias.reshape(-1)
  gb16 = jnp.broadcast_to(global_bias.astype(jnp.float32), (_L,))
  return _mf_sc(user, movie, uft, movie_factors, ub, mb, gb16)
